# Initial kernel scaffold; baseline (speedup 1.0000x reference)
import jax
import jax.numpy as jnp
from jax.experimental import pallas as pl


def kernel(inputs, emb_table):
    raise NotImplementedError("write your pallas kernel here")


# speedup vs baseline: 1.1311x; 1.1311x over previous
import jax
import jax.numpy as jnp
from jax import lax
from jax.experimental import pallas as pl
from jax.experimental.pallas import tpu as pltpu
from jax.experimental.pallas import tpu_sc as plsc

NODES_NUM = 1000
EMB_SIZE = 128
BATCH = 4096
SEQ = 50

NC, NS = 2, 16
NW = NC * NS
TOTAL = BATCH * SEQ
PER_W = TOTAL // NW
CHUNK = 128
NCHUNK = PER_W // CHUNK
OUT_W = EMB_SIZE + 2


def _sc_body(inp_hbm, table_hbm, out_hbm, inp_v, idx_v, rows_v, out_v, sem):
    wid = lax.axis_index("s") * NC + lax.axis_index("c")
    lanes = lax.iota(jnp.int32, 16)

    @pl.loop(0, NCHUNK)
    def _chunk(g):
        base = wid * PER_W + g * CHUNK
        pltpu.sync_copy(inp_hbm.at[pl.ds(base * 3, CHUNK * 3)], inp_v)
        for i in range(CHUNK // 16):
            fl = lanes * 3 + (i * 48)
            flat0 = (lanes + i * 16) * OUT_W
            idx_v[0, pl.ds(i * 16, 16)] = plsc.load_gather(
                inp_v, [fl]).astype(jnp.int32)
            plsc.store_scatter(
                out_v, [flat0], plsc.load_gather(inp_v, [fl + 1]))
            plsc.store_scatter(
                out_v, [flat0 + (OUT_W - 1)], plsc.load_gather(inp_v, [fl + 2]))
        pltpu.async_copy(table_hbm.at[idx_v.at[0]], rows_v, sem).wait()

        @pl.loop(0, CHUNK)
        def _row(r):
            dst = r * OUT_W + 1
            for j in range(EMB_SIZE // 16):
                out_v[pl.ds(dst + j * 16, 16)] = rows_v[r, pl.ds(j * 16, 16)]

        pltpu.sync_copy(out_v, out_hbm.at[pl.ds(base * OUT_W, CHUNK * OUT_W)])


@jax.jit
def kernel(inputs, emb_table):
    inp_flat = inputs.reshape(TOTAL * 3)
    mesh = plsc.VectorSubcoreMesh(core_axis_name="c", subcore_axis_name="s")
    out = pl.kernel(
        _sc_body,
        out_type=jax.ShapeDtypeStruct((TOTAL * OUT_W,), jnp.float32),
        mesh=mesh,
        scratch_types=[
            pltpu.VMEM((CHUNK * 3,), jnp.float32),
            pltpu.VMEM((1, CHUNK), jnp.int32),
            pltpu.VMEM((CHUNK, EMB_SIZE), jnp.float32),
            pltpu.VMEM((CHUNK * OUT_W,), jnp.float32),
            pltpu.SemaphoreType.DMA,
        ],
        compiler_params=pltpu.CompilerParams(
            use_tc_tiling_on_sc=False, needs_layout_passes=False),
    )(inp_flat, emb_table)
    return out.reshape(BATCH, SEQ, OUT_W)
